# baseline (speedup 1.0000x reference)
import functools

import jax
import jax.numpy as jnp
from jax import lax
from jax.experimental import pallas as pl
from jax.experimental.pallas import tpu as pltpu
from jax.experimental.pallas import tpu_sc as plsc


def _sc_gather(emb_flat, lin16, idx, lidx, window):
    ni = idx.shape[0]
    d = emb_flat.shape[1]
    idx2 = idx.reshape(1, ni)
    lidx2 = lidx.reshape(1, ni)
    mesh = plsc.VectorSubcoreMesh(core_axis_name="c", subcore_axis_name="s")

    @functools.partial(
        pl.kernel,
        out_type=(
            jax.ShapeDtypeStruct((ni, d), emb_flat.dtype),
            jax.ShapeDtypeStruct((ni, d), lin16.dtype),
        ),
        mesh=mesh,
        compiler_params=pltpu.CompilerParams(use_tc_tiling_on_sc=False),
    )
    def k(emb_hbm, lin_hbm, i_hbm, li_hbm, oe_hbm, ol_hbm):
        def body(i_vmem, li_vmem, oe_vmem, ol_vmem):
            pltpu.sync_copy(emb_hbm.at[i_vmem.at[0]], oe_vmem)
            pltpu.sync_copy(lin_hbm.at[li_vmem.at[0]], ol_vmem)

        pltpu.emit_pipeline(
            body,
            grid=(ni // window,),
            in_specs=[
                pl.BlockSpec((1, window), index_map=lambda i: (0, i)),
                pl.BlockSpec((1, window), index_map=lambda i: (0, i)),
            ],
            out_specs=[
                pl.BlockSpec((window, d), index_map=lambda i: (i, 0)),
                pl.BlockSpec((window, d), index_map=lambda i: (i, 0)),
            ],
            core_axis_name=("c", "s"),
            dimension_semantics=(pltpu.PARALLEL,),
        )(i_hbm, li_hbm, oe_hbm, ol_hbm)

    return k(emb_flat, lin16, idx2, lidx2)


def _dense_body(xe_ref, lr_ref, xm_ref, w1_ref, b1_ref, w2_ref, b2_ref,
                w3_ref, bb_ref, s_ref, e_ref, o_ref):
    xe = xe_ref[...]
    lane = lax.broadcasted_iota(jnp.int32, lr_ref.shape, 1) % 16
    xm416 = jnp.dot(xm_ref[...], e_ref[...], preferred_element_type=jnp.float32)
    mask = (lane.astype(jnp.float32) == xm416).astype(jnp.float32)
    lin_sum = jnp.sum(lr_ref[...] * mask, axis=1, keepdims=True)
    s = jnp.dot(xe, s_ref[...], preferred_element_type=jnp.float32)
    fm = 0.5 * (jnp.sum(s * s, axis=1, keepdims=True)
                - jnp.sum(xe * xe, axis=1, keepdims=True))
    h = jnp.dot(xe, w1_ref[...], preferred_element_type=jnp.float32)
    h = jnp.maximum(h + b1_ref[...], 0.0)
    h = jnp.dot(h, w2_ref[...], preferred_element_type=jnp.float32)
    h = jnp.maximum(h + b2_ref[...], 0.0)
    deep = jnp.dot(h, w3_ref[...], preferred_element_type=jnp.float32)
    o_ref[...] = jax.nn.sigmoid(lin_sum + fm + deep + bb_ref[...])


def _dense(xe, lr, xm, w1, b1, w2, b2, w3, bb, s, e, blk):
    b = xe.shape[0]
    grid = (b // blk,)
    full = lambda a: pl.BlockSpec(a.shape, lambda i: (0, 0))
    return pl.pallas_call(
        _dense_body,
        grid=grid,
        in_specs=[
            pl.BlockSpec((blk, xe.shape[1]), lambda i: (i, 0)),
            pl.BlockSpec((blk, lr.shape[1]), lambda i: (i, 0)),
            pl.BlockSpec((blk, xm.shape[1]), lambda i: (i, 0)),
            full(w1), full(b1), full(w2), full(b2), full(w3), full(bb),
            full(s), full(e),
        ],
        out_specs=pl.BlockSpec((blk, 1), lambda i: (i, 0)),
        out_shape=jax.ShapeDtypeStruct((b, 1), jnp.float32),
    )(xe, lr, xm, w1, b1, w2, b2, w3, bb, s, e)


def kernel(x, emb_tables, lin_tables, linear_bias, W1, b1, W2, b2, W3, b3):
    batch, nf = x.shape
    _, vocab, d = emb_tables.shape

    emb_flat = emb_tables.reshape(nf * vocab, d)
    lin16 = lin_tables.reshape(nf * (vocab // d), d)
    offs = (jnp.arange(nf, dtype=jnp.int32) * vocab)[None, :]
    idx = (x + offs).reshape(batch * nf)
    loffs = (jnp.arange(nf, dtype=jnp.int32) * (vocab // d))[None, :]
    lidx = ((x // d) + loffs).reshape(batch * nf)
    xmod = (x % d).astype(jnp.float32)

    rows, lrows = _sc_gather(emb_flat, lin16, idx, lidx, window=512)

    xe = rows.reshape(batch, nf * d)
    lr = lrows.reshape(batch, nf * d)

    s = jnp.tile(jnp.eye(d, dtype=jnp.float32), (nf, 1))
    e = jnp.repeat(jnp.eye(nf, dtype=jnp.float32), d, axis=1)
    bb = (b3 + linear_bias).reshape(1, 1)
    return _dense(xe, lr, xmod, W1, b1.reshape(1, -1), W2, b2.reshape(1, -1),
                  W3, bb, s, e, blk=2048)


# speedup vs baseline: 4.7431x; 4.7431x over previous
import functools

import jax
import jax.numpy as jnp
from jax import lax
from jax.experimental import pallas as pl
from jax.experimental.pallas import tpu as pltpu
from jax.experimental.pallas import tpu_sc as plsc

_F, _V, _D, _B = 26, 100000, 16, 16384
_CHUNK = 4096


def _sc_plane_gather(embT, linP, xT):
    mesh = plsc.VectorSubcoreMesh(core_axis_name="c", subcore_axis_name="s")

    @functools.partial(
        pl.kernel,
        out_type=(
            jax.ShapeDtypeStruct((52, 8, _B), jnp.float32),
            jax.ShapeDtypeStruct((4, 8, _B), jnp.float32),
        ),
        mesh=mesh,
        scratch_types=[
            pltpu.VMEM((_V,), jnp.float32),
            pltpu.VMEM((_CHUNK,), jnp.int32),
            pltpu.VMEM((_CHUNK,), jnp.float32),
        ],
        compiler_params=pltpu.CompilerParams(
            use_tc_tiling_on_sc=True, needs_layout_passes=False),
    )
    def k(embT_hbm, lin_hbm, xT_hbm, oe_hbm, ol_hbm, plane_v, idx_v, outv):
        cid = lax.axis_index("c")
        sid = lax.axis_index("s")
        half = sid // 8
        lane8 = sid % 8

        def gather_plane(f, out_row):
            @pl.loop(0, _B // _CHUNK)
            def _(h):
                pltpu.sync_copy(xT_hbm.at[f, pl.ds(h * _CHUNK, _CHUNK)], idx_v)

                @pl.loop(0, _CHUNK // 16)
                def _(i):
                    vals = plsc.load_gather(
                        plane_v, [idx_v[pl.ds(i * 16, 16)]])
                    outv[pl.ds(i * 16, 16)] = vals

                pltpu.sync_copy(outv, out_row.at[pl.ds(h * _CHUNK, _CHUNK)])

        @pl.loop(0, 13)
        def _(r):
            group = cid * 26 + r * 2 + half
            p = group * 8 + lane8
            f = p // _D
            dd = p % _D
            pltpu.sync_copy(embT_hbm.at[f, dd], plane_v)
            gather_plane(f, oe_hbm.at[group, lane8])

        group = cid * 2 + half
        f = group * 8 + lane8

        @pl.when(f < _F)
        def _():
            pltpu.sync_copy(lin_hbm.at[f], plane_v)
            gather_plane(f, ol_hbm.at[group, lane8])

        @pl.when(f >= _F)
        def _():
            @pl.loop(0, _CHUNK // 16)
            def _(i):
                outv[pl.ds(i * 16, 16)] = jnp.zeros((16,), jnp.float32)

            @pl.loop(0, _B // _CHUNK)
            def _(h):
                pltpu.sync_copy(
                    outv, ol_hbm.at[group, lane8, pl.ds(h * _CHUNK, _CHUNK)])

    return k(embT, linP, xT)


def _dense_body(xe_ref, lt_ref, w1t_ref, b1_ref, w2t_ref, b2_ref, w3t_ref,
                bb_ref, st_ref, o_ref):
    xeT = xe_ref[...]
    lin_sum = jnp.sum(lt_ref[...], axis=0, keepdims=True)
    sT = jnp.dot(st_ref[...], xeT, preferred_element_type=jnp.float32)
    fm = 0.5 * (jnp.sum(sT * sT, axis=0, keepdims=True)
                - jnp.sum(xeT * xeT, axis=0, keepdims=True))
    h = jnp.dot(w1t_ref[...], xeT, preferred_element_type=jnp.float32)
    h = jnp.maximum(h + b1_ref[...], 0.0)
    h = jnp.dot(w2t_ref[...], h, preferred_element_type=jnp.float32)
    h = jnp.maximum(h + b2_ref[...], 0.0)
    deep = jnp.dot(w3t_ref[...], h, preferred_element_type=jnp.float32)
    o_ref[...] = jax.nn.sigmoid(lin_sum + fm + deep + bb_ref[...])


def _dense(xeT, linT, w1t, b1c, w2t, b2c, w3t, bb, sT, bs):
    grid = (_B // bs,)
    full = lambda a: pl.BlockSpec(a.shape, lambda i: (0, 0))
    return pl.pallas_call(
        _dense_body,
        grid=grid,
        in_specs=[
            pl.BlockSpec((416, bs), lambda i: (0, i)),
            pl.BlockSpec((32, bs), lambda i: (0, i)),
            full(w1t), full(b1c), full(w2t), full(b2c), full(w3t),
            full(bb), full(sT),
        ],
        out_specs=pl.BlockSpec((1, bs), lambda i: (0, i)),
        out_shape=jax.ShapeDtypeStruct((1, _B), jnp.float32),
    )(xeT, linT, w1t, b1c, w2t, b2c, w3t, bb, sT)


def kernel(x, emb_tables, lin_tables, linear_bias, W1, b1, W2, b2, W3, b3):
    embT = jnp.transpose(emb_tables, (0, 2, 1))
    linP = jnp.transpose(lin_tables, (0, 2, 1))[:, 0, :]
    xT = x.T

    xeT4, linT4 = _sc_plane_gather(embT, linP, xT)
    xeT = xeT4.reshape(416, _B)
    linT = linT4.reshape(32, _B)

    sT = jnp.tile(jnp.eye(_D, dtype=jnp.float32), (1, _F))
    bb = (b3 + linear_bias).reshape(1, 1)
    out = _dense(xeT, linT, W1.T, b1.reshape(-1, 1), W2.T, b2.reshape(-1, 1),
                 W3.T, bb, sT, bs=2048)
    return out.reshape(_B, 1)
